# Initial kernel scaffold; baseline (speedup 1.0000x reference)
import jax
import jax.numpy as jnp
from jax.experimental import pallas as pl


def kernel(x, edge_index, W1l, W1r, W2l, W2r, Wo):
    raise NotImplementedError("write your pallas kernel here")


# speedup vs baseline: 6.4532x; 6.4532x over previous
import functools

import jax
import jax.numpy as jnp
from jax import lax
from jax.experimental import pallas as pl
from jax.experimental.pallas import tpu as pltpu
from jax.experimental.pallas import tpu_sc as plsc

N = 10000
E = 320000
D_IN = 128
D_HID = 256
D_OUT = 40
D1 = 144
D2 = 48
NC = 2
NS = 16
NW = NC * NS
EPT = E // NW
K = 80
G = EPT // K
ROWCHUNK = 400
NCHUNK = N // ROWCHUNK
BR = 400


def _make_seg_sum(D):
  def body(table, src, dst, zeros, out, acc, sidx, didx, rows, sem):
    c = lax.axis_index("c")
    s = lax.axis_index("s")
    wid = s * NC + c

    pltpu.sync_copy(zeros.at[pl.ds(s * ROWCHUNK, ROWCHUNK)],
                    acc.at[pl.ds(s * ROWCHUNK, ROWCHUNK)])

    @pl.when(s + NS < NCHUNK)
    def _():
      pltpu.sync_copy(zeros.at[pl.ds((s + NS) * ROWCHUNK, ROWCHUNK)],
                      acc.at[pl.ds((s + NS) * ROWCHUNK, ROWCHUNK)])

    plsc.subcore_barrier()

    base = wid * EPT

    def step(g, carry):
      off = base + g * K
      pltpu.sync_copy(src.at[pl.ds(off, K)], sidx)
      pltpu.sync_copy(dst.at[pl.ds(off, K)], didx)
      pltpu.async_copy(table.at[sidx], rows, sem).wait()
      pltpu.sync_copy(rows, acc.at[didx], add=True)
      return carry

    lax.fori_loop(0, G, step, 0)
    plsc.subcore_barrier()

    pltpu.sync_copy(acc.at[pl.ds(s * ROWCHUNK, ROWCHUNK)],
                    out.at[pl.ds(c * N + s * ROWCHUNK, ROWCHUNK)])

    @pl.when(s + NS < NCHUNK)
    def _():
      pltpu.sync_copy(acc.at[pl.ds((s + NS) * ROWCHUNK, ROWCHUNK)],
                      out.at[pl.ds(c * N + (s + NS) * ROWCHUNK, ROWCHUNK)])

  mesh = plsc.VectorSubcoreMesh(core_axis_name="c", subcore_axis_name="s")
  return pl.kernel(
      body,
      out_type=jax.ShapeDtypeStruct((NC * N, D), jnp.float32),
      mesh=mesh,
      scratch_types=[
          pltpu.VMEM_SHARED((N, D), jnp.float32),
          pltpu.VMEM((K,), jnp.int32),
          pltpu.VMEM((K,), jnp.int32),
          pltpu.VMEM((K, D), jnp.float32),
          pltpu.SemaphoreType.DMA,
      ],
      compiler_params=pltpu.CompilerParams(use_tc_tiling_on_sc=False),
  )


def _fold_body(w2l_ref, w2r_ref, wop_ref, ml_ref, mr_ref):
  ml_ref[...] = jnp.dot(w2l_ref[...], wop_ref[...],
                        preferred_element_type=jnp.float32)
  mr_ref[...] = jnp.dot(w2r_ref[...], wop_ref[...],
                        preferred_element_type=jnp.float32)


def _mid_body(part_ref, x_ref, w1l_ref, w1r_ref, ml_ref, mr_ref,
              p_ref, qi_ref):
  sfull = part_ref[0] + part_ref[1]
  agg = sfull[:, :D_IN]
  cnt = sfull[:, D_IN:D_IN + 1]
  inv = 1.0 / jnp.maximum(cnt, 1.0)
  h = (jnp.dot(agg * inv, w1l_ref[...], preferred_element_type=jnp.float32)
       + jnp.dot(x_ref[...], w1r_ref[...], preferred_element_type=jnp.float32))
  h = jnp.maximum(h, 0.0)
  p_ref[...] = jnp.dot(h, ml_ref[...], preferred_element_type=jnp.float32)
  col = lax.broadcasted_iota(jnp.int32, (BR, D2), 1)
  qi_ref[...] = (jnp.dot(h, mr_ref[...], preferred_element_type=jnp.float32)
                 + jnp.where(col == D_OUT, inv, 0.0))


def _out_body(part2_ref, qi_ref, o_ref):
  s2 = part2_ref[0] + part2_ref[1]
  inv = qi_ref[:, D_OUT:D_OUT + 1]
  logits = s2[:, :D_OUT] * inv + qi_ref[:, :D_OUT]
  m = jnp.max(logits, axis=1, keepdims=True)
  e = jnp.exp(logits - m)
  lse = jnp.log(jnp.sum(e, axis=1, keepdims=True))
  o_ref[...] = logits - m - lse


_seg1 = _make_seg_sum(D1)
_seg2 = _make_seg_sum(D2)

_fold = pl.pallas_call(
    _fold_body,
    out_shape=(jax.ShapeDtypeStruct((D_HID, D2), jnp.float32),
               jax.ShapeDtypeStruct((D_HID, D2), jnp.float32)),
)

_mid = pl.pallas_call(
    _mid_body,
    grid=(N // BR,),
    in_specs=[
        pl.BlockSpec((2, BR, D1), lambda i: (0, i, 0)),
        pl.BlockSpec((BR, D_IN), lambda i: (i, 0)),
        pl.BlockSpec((D_IN, D_HID), lambda i: (0, 0)),
        pl.BlockSpec((D_IN, D_HID), lambda i: (0, 0)),
        pl.BlockSpec((D_HID, D2), lambda i: (0, 0)),
        pl.BlockSpec((D_HID, D2), lambda i: (0, 0)),
    ],
    out_specs=(pl.BlockSpec((BR, D2), lambda i: (i, 0)),
               pl.BlockSpec((BR, D2), lambda i: (i, 0))),
    out_shape=(jax.ShapeDtypeStruct((N, D2), jnp.float32),
               jax.ShapeDtypeStruct((N, D2), jnp.float32)),
)

_outk = pl.pallas_call(
    _out_body,
    grid=(N // BR,),
    in_specs=[
        pl.BlockSpec((2, BR, D2), lambda i: (0, i, 0)),
        pl.BlockSpec((BR, D2), lambda i: (i, 0)),
    ],
    out_specs=pl.BlockSpec((BR, D_OUT), lambda i: (i, 0)),
    out_shape=jax.ShapeDtypeStruct((N, D_OUT), jnp.float32),
)


@jax.jit
def kernel(x, edge_index, W1l, W1r, W2l, W2r, Wo):
  src = edge_index[0].astype(jnp.int32)
  dst = edge_index[1].astype(jnp.int32)
  xaug = jnp.concatenate(
      [x, jnp.ones((N, 1), jnp.float32), jnp.zeros((N, D1 - D_IN - 1), jnp.float32)],
      axis=1)
  wop = jnp.pad(Wo, ((0, 0), (0, D2 - D_OUT)))
  ml, mr = _fold(W2l, W2r, wop)
  zeros1 = jnp.zeros((N, D1), jnp.float32)
  part1 = _seg1(xaug, src, dst, zeros1).reshape(NC, N, D1)
  p, qi = _mid(part1, x, W1l, W1r, ml, mr)
  zeros2 = jnp.zeros((N, D2), jnp.float32)
  part2 = _seg2(p, src, dst, zeros2).reshape(NC, N, D2)
  return _outk(part2, qi)
